# baseline (speedup 1.0000x reference)
import functools
import math

import jax
import jax.numpy as jnp
from jax import lax
from jax.experimental import pallas as pl
from jax.experimental.pallas import tpu as pltpu
from jax.experimental.pallas import tpu_sc as plsc

_D = 64
_SCALE = math.sqrt(_D)
_LANES = 16
_IDX_W = 128
_CHUNK = 512
_K = _CHUNK // _IDX_W


def _embed_sc(x2d, table):
    n_rows = x2d.shape[0]
    B = n_rows * _IDX_W
    info = plsc.get_sparse_core_info()
    nw = info.num_cores * info.num_subcores
    per_w = B // nw
    n_chunks = per_w // _CHUNK
    rows_per_chunk = _CHUNK // _IDX_W

    mesh = plsc.VectorSubcoreMesh(core_axis_name="c", subcore_axis_name="s")

    @functools.partial(
        pl.kernel,
        out_type=jax.ShapeDtypeStruct((B, _D), jnp.float32),
        mesh=mesh,
        scratch_types=[
            pltpu.VMEM((_K, _IDX_W), jnp.int32),
            pltpu.VMEM((_CHUNK, _D), jnp.float32),
            pltpu.SemaphoreType.DMA,
        ],
        compiler_params=pltpu.CompilerParams(use_tc_tiling_on_sc=False),
    )
    def k(x_hbm, tab_hbm, out_hbm, idx_v, rows_v, sem):
        wid = lax.axis_index("s") * info.num_cores + lax.axis_index("c")
        row_base_w = wid * (per_w // _IDX_W)

        def chunk_body(g, carry):
            row_base = row_base_w + g * rows_per_chunk
            pltpu.sync_copy(x_hbm.at[pl.ds(row_base, rows_per_chunk)], idx_v)
            copies = [
                pltpu.async_copy(
                    tab_hbm.at[idx_v.at[j]],
                    rows_v.at[pl.ds(j * _IDX_W, _IDX_W)],
                    sem,
                )
                for j in range(_K)
            ]
            for c in copies:
                c.wait()

            def scale_body(i, carry2):
                r = i * 4
                for dr in range(4):
                    for j in range(_D // _LANES):
                        sl = (r + dr, pl.ds(j * _LANES, _LANES))
                        rows_v[sl] = rows_v[sl] * _SCALE
                return carry2

            lax.fori_loop(0, _CHUNK // 4, scale_body, 0, unroll=False)
            pltpu.sync_copy(
                rows_v, out_hbm.at[pl.ds(row_base * _IDX_W, _CHUNK)]
            )
            return carry

        lax.fori_loop(0, n_chunks, chunk_body, 0, unroll=False)

    return k(x2d, table)


def kernel(x, table):
    b, t = x.shape
    x2d = x.reshape(-1, _IDX_W).astype(jnp.int32)
    out = _embed_sc(x2d, table)
    return out.reshape(b, t, _D)


# speedup vs baseline: 1.0817x; 1.0817x over previous
import functools
import math

import jax
import jax.numpy as jnp
from jax import lax
from jax.experimental import pallas as pl
from jax.experimental.pallas import tpu as pltpu
from jax.experimental.pallas import tpu_sc as plsc

_D = 64
_SCALE = math.sqrt(_D)
_LANES = 16
_IDX_W = 128
_CHUNK = 256
_K = _CHUNK // _IDX_W
_NBUF = 4
_AHEAD = 2


def _embed_sc(x2d, table):
    n_rows = x2d.shape[0]
    B = n_rows * _IDX_W
    info = plsc.get_sparse_core_info()
    nw = info.num_cores * info.num_subcores
    per_w = B // nw
    n_chunks = per_w // _CHUNK
    rpc = _CHUNK // _IDX_W
    assert per_w % _CHUNK == 0 and n_chunks % _NBUF == 0 and n_chunks >= 2 * _NBUF

    mesh = plsc.VectorSubcoreMesh(core_axis_name="c", subcore_axis_name="s")

    @functools.partial(
        pl.kernel,
        out_type=jax.ShapeDtypeStruct((B, _D), jnp.float32),
        mesh=mesh,
        scratch_types=[
            pltpu.VMEM((_NBUF, _K, _IDX_W), jnp.int32),
            pltpu.VMEM((_NBUF, _CHUNK, _D), jnp.float32),
            pltpu.SemaphoreType.DMA((_NBUF,)),
            pltpu.SemaphoreType.DMA((_NBUF,)),
        ],
        compiler_params=pltpu.CompilerParams(use_tc_tiling_on_sc=False),
    )
    def k(x_hbm, tab_hbm, out_hbm, idx_v, rows_v, sem_g, sem_o):
        wid = lax.axis_index("s") * info.num_cores + lax.axis_index("c")
        row_base_w = wid * (per_w // _IDX_W)

        def fire_gather(g, b):
            pltpu.sync_copy(
                x_hbm.at[pl.ds(row_base_w + g * rpc, rpc)], idx_v.at[b]
            )
            for j in range(_K):
                pltpu.async_copy(
                    tab_hbm.at[idx_v.at[b, j]],
                    rows_v.at[b, pl.ds(j * _IDX_W, _IDX_W)],
                    sem_g.at[b],
                )

        def drain_gather(b):
            pltpu.make_async_copy(
                tab_hbm.at[pl.ds(0, _CHUNK)], rows_v.at[b], sem_g.at[b]
            ).wait()

        def scale(b):
            def body(i, c):
                r = i * 4
                for dr in range(4):
                    for j in range(_D // _LANES):
                        sl = (b, r + dr, pl.ds(j * _LANES, _LANES))
                        rows_v[sl] = rows_v[sl] * _SCALE
                return c

            lax.fori_loop(0, _CHUNK // 4, body, 0, unroll=False)

        def fire_out(g, b):
            pltpu.async_copy(
                rows_v.at[b],
                out_hbm.at[pl.ds((row_base_w + g * rpc) * _IDX_W, _CHUNK)],
                sem_o.at[b],
            )

        def drain_out(b):
            pltpu.make_async_copy(
                rows_v.at[b], out_hbm.at[pl.ds(0, _CHUNK)], sem_o.at[b]
            ).wait()

        for g in range(_AHEAD):
            fire_gather(g, g)

        for g in range(_AHEAD):
            b = g % _NBUF
            drain_gather(b)
            scale(b)
            fire_out(g, b)
            fire_gather(g + _AHEAD, (g + _AHEAD) % _NBUF)

        n_steady = n_chunks - 2 * _AHEAD
        assert n_steady % _NBUF == 0

        def super_step(s, carry):
            for p in range(_NBUF):
                g = _AHEAD + s * _NBUF + p
                b = (_AHEAD + p) % _NBUF
                drain_gather(b)
                scale(b)
                fire_out(g, b)
                drain_out(p)
                fire_gather(g + _AHEAD, p)
            return carry

        lax.fori_loop(0, n_steady // _NBUF, super_step, 0, unroll=False)

        for g in range(n_chunks - _AHEAD, n_chunks):
            b = g % _NBUF
            drain_gather(b)
            scale(b)
            fire_out(g, b)

        for b in range(_NBUF):
            drain_out(b)

    return k(x2d, table)


def kernel(x, table):
    b, t = x.shape
    x2d = x.reshape(-1, _IDX_W).astype(jnp.int32)
    out = _embed_sc(x2d, table)
    return out.reshape(b, t, _D)
